# baseline (speedup 1.0000x reference)
import functools

import jax
import jax.numpy as jnp
from jax import lax
from jax.experimental import pallas as pl
from jax.experimental.pallas import tpu as pltpu
from jax.experimental.pallas import tpu_sc as plsc

_NC, _NS, _L = 2, 16, 16
_NW = _NC * _NS


@functools.lru_cache(maxsize=None)
def _make_embed(B, T, V, D):
    TW = T // _NW
    C = TW // 2
    n_vregs = D // _L
    U = B * 2

    mesh = plsc.VectorSubcoreMesh(core_axis_name="c", subcore_axis_name="s")

    @functools.partial(
        pl.kernel,
        out_type=jax.ShapeDtypeStruct((B * T, D), jnp.float32),
        mesh=mesh,
        scratch_types=[
            pltpu.VMEM((C,), jnp.int32),
            pltpu.VMEM((C,), jnp.int32),
            pltpu.VMEM((C, D), jnp.float32),
            pltpu.VMEM((C, D), jnp.float32),
            pltpu.VMEM((TW, D), jnp.float32),
            pltpu.SemaphoreType.DMA,
            pltpu.SemaphoreType.DMA,
            pltpu.SemaphoreType.DMA,
            pltpu.SemaphoreType.DMA,
            pltpu.SemaphoreType.DMA,
        ],
    )
    def embed(idx_hbm, wte_hbm, wpe_hbm, out_hbm,
              idx0, idx1, rows0, rows1, wpe_v,
              gsem0, gsem1, wsem0, wsem1, psem):
        idx_v = (idx0, idx1)
        rows_v = (rows0, rows1)
        gsem = (gsem0, gsem1)
        wsem = (wsem0, wsem1)

        wid = lax.axis_index("s") * _NC + lax.axis_index("c")
        t0 = wid * TW
        wpe_cp = pltpu.async_copy(wpe_hbm.at[pl.ds(t0, TW)], wpe_v, psem)

        def unit_base(u):
            b, h = divmod(u, 2)
            return b * T + t0 + h * C, h * C

        def start_gather(u):
            p = u % 2
            base, _ = unit_base(u)
            pltpu.sync_copy(idx_hbm.at[pl.ds(base, C)], idx_v[p])
            return pltpu.async_copy(wte_hbm.at[idx_v[p]], rows_v[p], gsem[p])

        gathers = {0: start_gather(0)}
        writes = {}
        wpe_cp.wait()
        for u in range(U):
            p = u % 2
            if u + 1 < U:
                if u - 1 >= 0:
                    writes[u - 1].wait()
                gathers[u + 1] = start_gather(u + 1)
            gathers[u].wait()
            _, woff = unit_base(u)

            def row_body(r, carry):
                for c in range(n_vregs):
                    sl = pl.ds(c * _L, _L)
                    rows_v[p][r, sl] = rows_v[p][r, sl] + wpe_v[woff + r, sl]
                return carry

            lax.fori_loop(0, C, row_body, 0)
            base, _ = unit_base(u)
            writes[u] = pltpu.async_copy(rows_v[p], out_hbm.at[pl.ds(base, C)],
                                         wsem[p])
        writes[U - 2].wait()
        writes[U - 1].wait()

    return embed


def kernel(idx, wte, wpe):
    B, T = idx.shape
    V, D = wte.shape
    out = _make_embed(B, T, V, D)(idx.reshape(-1).astype(jnp.int32), wte, wpe)
    return out.reshape(B, T, D)
